# baseline (device time: 56238 ns/iter reference)
import jax
import jax.numpy as jnp
from jax import lax
from jax.experimental import pallas as pl
from jax.experimental.pallas import tpu as pltpu

B = 4
S = 512
S_OUT = 256
K = 512
N = 1024


def kernel(O, Wo):
    O2 = O.reshape(B, S, K)

    def body(o_ref, w_ref, out_ref, send_buf, recv_buf, send_sem, recv_sem):
        my_x = lax.axis_index("x")
        my_y = lax.axis_index("y")
        other_x = 1 - my_x

        barrier = pltpu.get_barrier_semaphore()
        pl.semaphore_signal(
            barrier, inc=1,
            device_id=(other_x, my_y),
            device_id_type=pl.DeviceIdType.MESH,
        )
        pl.semaphore_wait(barrier, 1)

        other_start = other_x * S_OUT
        for b in range(B):
            send_buf[b] = jnp.dot(
                o_ref[b, pl.ds(other_start, S_OUT), :],
                w_ref[...],
                preferred_element_type=jnp.float32,
            )

        rdma = pltpu.make_async_remote_copy(
            src_ref=send_buf,
            dst_ref=recv_buf,
            send_sem=send_sem,
            recv_sem=recv_sem,
            device_id=(other_x, my_y),
            device_id_type=pl.DeviceIdType.MESH,
        )
        rdma.start()

        my_start = my_x * S_OUT
        for b in range(B):
            out_ref[b] = jnp.dot(
                o_ref[b, pl.ds(my_start, S_OUT), :],
                w_ref[...],
                preferred_element_type=jnp.float32,
            )

        rdma.wait()
        for b in range(B):
            out_ref[b] = out_ref[b] + recv_buf[b]

    return pl.pallas_call(
        body,
        out_shape=jax.ShapeDtypeStruct((B, S_OUT, N), jnp.float32),
        in_specs=[
            pl.BlockSpec(memory_space=pltpu.VMEM),
            pl.BlockSpec(memory_space=pltpu.VMEM),
        ],
        out_specs=pl.BlockSpec(memory_space=pltpu.VMEM),
        scratch_shapes=[
            pltpu.VMEM((B, S_OUT, N), jnp.float32),
            pltpu.VMEM((B, S_OUT, N), jnp.float32),
            pltpu.SemaphoreType.DMA,
            pltpu.SemaphoreType.DMA,
        ],
        compiler_params=pltpu.CompilerParams(collective_id=0),
    )(O2, Wo)


# device time: 40499 ns/iter; 1.3886x vs baseline; 1.3886x over previous
import jax
import jax.numpy as jnp
from jax import lax
from jax.experimental import pallas as pl
from jax.experimental.pallas import tpu as pltpu

B = 4
S = 512
S_OUT = 256
SQ = 128
K = 512
N = 1024


def kernel(O, Wo):
    O2 = O.reshape(B, S, K)

    def body(o_ref, w_ref, out_ref, xsend_buf, xrecv_buf,
             xsend_sems, xrecv_sems, ysend_sems, yrecv_sems):
        my_x = lax.axis_index("x")
        my_y = lax.axis_index("y")
        ox = 1 - my_x
        oy = 1 - my_y

        barrier = pltpu.get_barrier_semaphore()
        pl.semaphore_signal(
            barrier, inc=1,
            device_id=(ox, my_y), device_id_type=pl.DeviceIdType.MESH,
        )
        pl.semaphore_signal(
            barrier, inc=1,
            device_id=(my_x, oy), device_id_type=pl.DeviceIdType.MESH,
        )
        pl.semaphore_wait(barrier, 2)

        my_q = my_x * S_OUT + my_y * SQ
        nb_q = ox * S_OUT + my_y * SQ
        loc = my_y * SQ

        x_rdmas = []
        for b in range(B):
            xsend_buf[b] = jnp.dot(
                o_ref[b, pl.ds(nb_q, SQ), :], w_ref[...],
                preferred_element_type=jnp.float32,
            )
            rdma = pltpu.make_async_remote_copy(
                src_ref=xsend_buf.at[b],
                dst_ref=xrecv_buf.at[b],
                send_sem=xsend_sems.at[b],
                recv_sem=xrecv_sems.at[b],
                device_id=(ox, my_y),
                device_id_type=pl.DeviceIdType.MESH,
            )
            rdma.start()
            x_rdmas.append(rdma)

        for b in range(B):
            out_ref[b, pl.ds(loc, SQ), :] = jnp.dot(
                o_ref[b, pl.ds(my_q, SQ), :], w_ref[...],
                preferred_element_type=jnp.float32,
            )

        y_rdmas = []
        for b in range(B):
            x_rdmas[b].wait()
            out_ref[b, pl.ds(loc, SQ), :] = (
                out_ref[b, pl.ds(loc, SQ), :] + xrecv_buf[b]
            )
            yr = pltpu.make_async_remote_copy(
                src_ref=out_ref.at[b, pl.ds(loc, SQ), :],
                dst_ref=out_ref.at[b, pl.ds(loc, SQ), :],
                send_sem=ysend_sems.at[b],
                recv_sem=yrecv_sems.at[b],
                device_id=(my_x, oy),
                device_id_type=pl.DeviceIdType.MESH,
            )
            yr.start()
            y_rdmas.append(yr)

        for b in range(B):
            y_rdmas[b].wait()

    return pl.pallas_call(
        body,
        out_shape=jax.ShapeDtypeStruct((B, S_OUT, N), jnp.float32),
        in_specs=[
            pl.BlockSpec(memory_space=pltpu.VMEM),
            pl.BlockSpec(memory_space=pltpu.VMEM),
        ],
        out_specs=pl.BlockSpec(memory_space=pltpu.VMEM),
        scratch_shapes=[
            pltpu.VMEM((B, SQ, N), jnp.float32),
            pltpu.VMEM((B, SQ, N), jnp.float32),
            pltpu.SemaphoreType.DMA((B,)),
            pltpu.SemaphoreType.DMA((B,)),
            pltpu.SemaphoreType.DMA((B,)),
            pltpu.SemaphoreType.DMA((B,)),
        ],
        compiler_params=pltpu.CompilerParams(collective_id=0),
    )(O2, Wo)


# device time: 37745 ns/iter; 1.4899x vs baseline; 1.0730x over previous
import jax
import jax.numpy as jnp
from jax import lax
from jax.experimental import pallas as pl
from jax.experimental.pallas import tpu as pltpu

B = 4
S = 512
S_OUT = 256
SQ = 128
K = 512
N = 1024
CPB = 2
CR = SQ // CPB
NC = B * CPB


def kernel(O, Wo):
    O2 = O.reshape(B, S, K)

    def body(o_ref, w_ref, out_ref, xsend_buf, xrecv_buf,
             xsend_sems, xrecv_sems, ysend_sems, yrecv_sems):
        my_x = lax.axis_index("x")
        my_y = lax.axis_index("y")
        ox = 1 - my_x
        oy = 1 - my_y

        barrier = pltpu.get_barrier_semaphore()
        pl.semaphore_signal(
            barrier, inc=1,
            device_id=(ox, my_y), device_id_type=pl.DeviceIdType.MESH,
        )
        pl.semaphore_signal(
            barrier, inc=1,
            device_id=(my_x, oy), device_id_type=pl.DeviceIdType.MESH,
        )
        pl.semaphore_wait(barrier, 2)

        my_q = my_x * S_OUT + my_y * SQ
        nb_q = ox * S_OUT + my_y * SQ
        loc = my_y * SQ

        x_rdmas = []
        for c in range(NC):
            b, half = divmod(c, CPB)
            xsend_buf[c] = jnp.dot(
                o_ref[b, pl.ds(nb_q + half * CR, CR), :], w_ref[...],
                preferred_element_type=jnp.float32,
            )
            rdma = pltpu.make_async_remote_copy(
                src_ref=xsend_buf.at[c],
                dst_ref=xrecv_buf.at[c],
                send_sem=xsend_sems.at[c],
                recv_sem=xrecv_sems.at[c],
                device_id=(ox, my_y),
                device_id_type=pl.DeviceIdType.MESH,
            )
            rdma.start()
            x_rdmas.append(rdma)

        for b in range(B):
            out_ref[b, pl.ds(loc, SQ), :] = jnp.dot(
                o_ref[b, pl.ds(my_q, SQ), :], w_ref[...],
                preferred_element_type=jnp.float32,
            )

        y_rdmas = []
        for c in range(NC):
            b, half = divmod(c, CPB)
            row = loc + half * CR
            x_rdmas[c].wait()
            out_ref[b, pl.ds(row, CR), :] = (
                out_ref[b, pl.ds(row, CR), :] + xrecv_buf[c]
            )
            yr = pltpu.make_async_remote_copy(
                src_ref=out_ref.at[b, pl.ds(row, CR), :],
                dst_ref=out_ref.at[b, pl.ds(row, CR), :],
                send_sem=ysend_sems.at[c],
                recv_sem=yrecv_sems.at[c],
                device_id=(my_x, oy),
                device_id_type=pl.DeviceIdType.MESH,
            )
            yr.start()
            y_rdmas.append(yr)

        for c in range(NC):
            y_rdmas[c].wait()

    return pl.pallas_call(
        body,
        out_shape=jax.ShapeDtypeStruct((B, S_OUT, N), jnp.float32),
        in_specs=[
            pl.BlockSpec(memory_space=pltpu.VMEM),
            pl.BlockSpec(memory_space=pltpu.VMEM),
        ],
        out_specs=pl.BlockSpec(memory_space=pltpu.VMEM),
        scratch_shapes=[
            pltpu.VMEM((NC, CR, N), jnp.float32),
            pltpu.VMEM((NC, CR, N), jnp.float32),
            pltpu.SemaphoreType.DMA((NC,)),
            pltpu.SemaphoreType.DMA((NC,)),
            pltpu.SemaphoreType.DMA((NC,)),
            pltpu.SemaphoreType.DMA((NC,)),
        ],
        compiler_params=pltpu.CompilerParams(collective_id=0),
    )(O2, Wo)
